# Initial kernel scaffold; baseline (speedup 1.0000x reference)
import jax
import jax.numpy as jnp
from jax.experimental import pallas as pl


def kernel(x, embeddings):
    raise NotImplementedError("write your pallas kernel here")


# speedup vs baseline: 35.4569x; 35.4569x over previous
import functools

import jax
import jax.numpy as jnp
from jax import lax
from jax.experimental import pallas as pl
from jax.experimental.pallas import tpu as pltpu
from jax.experimental.pallas import tpu_sc as plsc

_NUM_EMB = 8192
_DIM = 32
_K = 8
_BETA = 0.25
_ROWS = 256

_SC_CORES = 2
_SC_SUBCORES = 16
_SC_WORKERS = _SC_CORES * _SC_SUBCORES


def _topk_body(x_ref, emb_ref, idx_ref, loss_ref, *, n_rows_total):
    i = pl.program_id(0)
    xb = x_ref[...]
    emb = emb_ref[...]
    sim = jnp.dot(xb, emb, preferred_element_type=jnp.float32)
    x2 = jnp.sum(xb * xb, axis=1, keepdims=True)
    e2 = jnp.sum(emb * emb, axis=0, keepdims=True)
    dist = x2 + e2 - 2.0 * sim
    lanes = lax.broadcasted_iota(jnp.int32, dist.shape, 1)
    total = jnp.zeros((), jnp.float32)
    idx_cols = []
    for _ in range(_K):
        m = jnp.min(dist, axis=1, keepdims=True)
        sel = jnp.where(dist == m, lanes, jnp.int32(2**30))
        idx = jnp.min(sel, axis=1, keepdims=True)
        idx_cols.append(idx)
        total = total + jnp.sum(m)
        dist = jnp.where(lanes == idx, jnp.float32(jnp.inf), dist)
    idx_ref[...] = jnp.concatenate(idx_cols, axis=1)

    scale = jnp.float32((1.0 + _BETA) / (n_rows_total * _K * _DIM))

    @pl.when(i == 0)
    def _init():
        loss_ref[0, 0] = 0.0

    loss_ref[0, 0] += total * scale


def _topk(flat, embeddings):
    n = flat.shape[0]
    grid = (n // _ROWS,)
    body = functools.partial(_topk_body, n_rows_total=n)
    return pl.pallas_call(
        body,
        grid=grid,
        in_specs=[
            pl.BlockSpec((_ROWS, _DIM), lambda i: (i, 0)),
            pl.BlockSpec((_DIM, _NUM_EMB), lambda i: (0, 0)),
        ],
        out_specs=[
            pl.BlockSpec((_ROWS, _K), lambda i: (i, 0)),
            pl.BlockSpec((1, 1), lambda i: (0, 0), memory_space=pltpu.SMEM),
        ],
        out_shape=[
            jax.ShapeDtypeStruct((n, _K), jnp.int32),
            jax.ShapeDtypeStruct((1, 1), jnp.float32),
        ],
    )(flat, embeddings)


def _sc_gather(table, idx_flat):
    b = idx_flat.shape[0]
    b_per_w = b // _SC_WORKERS
    mesh = plsc.VectorSubcoreMesh(core_axis_name="c", subcore_axis_name="s")

    @functools.partial(
        pl.kernel,
        out_type=jax.ShapeDtypeStruct((b, _DIM), jnp.float32),
        mesh=mesh,
        scratch_types=[
            pltpu.VMEM((b_per_w,), jnp.int32),
            pltpu.VMEM((b_per_w, _DIM), jnp.float32),
            pltpu.SemaphoreType.DMA,
        ],
        compiler_params=pltpu.CompilerParams(use_tc_tiling_on_sc=False),
    )
    def gk(table_hbm, idx_hbm, out_hbm, idx_v, rows_v, sem):
        wid = lax.axis_index("s") * _SC_CORES + lax.axis_index("c")
        base = wid * b_per_w
        pltpu.sync_copy(idx_hbm.at[pl.ds(base, b_per_w)], idx_v)
        pltpu.async_copy(table_hbm.at[idx_v], rows_v, sem).wait()
        pltpu.sync_copy(rows_v, out_hbm.at[pl.ds(base, b_per_w)])

    return gk(table, idx_flat)


def kernel(x, embeddings):
    bsz, t, d = x.shape
    flat = x.reshape(-1, d)
    idx, loss = _topk(flat, embeddings)
    table = embeddings.T
    q = _sc_gather(table, idx.reshape(-1))
    quantized = q.reshape(bsz, t, _K, d)
    return quantized, loss[0, 0]
